# baseline (speedup 1.0000x reference)
import jax
import jax.numpy as jnp
from jax.experimental import pallas as pl
from jax.experimental.pallas import tpu as pltpu


_DEG, _N, _D = 16, 10000, 256
_NB = 400


def _reduce_body(ft_ref, out_ref):
    out_ref[...] = jnp.sum(ft_ref[...], axis=0)


def kernel(a, ft):
    del a
    out = pl.pallas_call(
        _reduce_body,
        grid=(_N // _NB,),
        in_specs=[pl.BlockSpec((_DEG, _NB, _D), lambda i: (0, i, 0))],
        out_specs=pl.BlockSpec((_NB, _D), lambda i: (i, 0)),
        out_shape=jax.ShapeDtypeStruct((_N, _D), jnp.float32),
        compiler_params=pltpu.CompilerParams(dimension_semantics=("parallel",)),
    )(ft)
    return out


# speedup vs baseline: 1.0009x; 1.0009x over previous
import jax
import jax.numpy as jnp
from jax.experimental import pallas as pl


_DEG, _N, _D = 16, 10000, 256
_NB = 400


def _reduce_body(ft_ref, out_ref):
    out_ref[...] = jnp.sum(ft_ref[...], axis=0)


def kernel(a, ft):
    del a
    out = pl.pallas_call(
        _reduce_body,
        grid=(_N // _NB,),
        in_specs=[pl.BlockSpec((_DEG, _NB, _D), lambda i: (0, i, 0))],
        out_specs=pl.BlockSpec((_NB, _D), lambda i: (i, 0)),
        out_shape=jax.ShapeDtypeStruct((_N, _D), jnp.float32),
    )(ft)
    return out
